# initial kernel scaffold (unmeasured)
import jax
import jax.numpy as jnp
from jax import lax
from jax.experimental import pallas as pl
from jax.experimental.pallas import tpu as pltpu

N_DEV = 4


def _body(q_ref, k_ref, v_ref, out_ref, comm_ref, send_sems, recv_sems):
    my = lax.axis_index("i")
    left = lax.rem(my + N_DEV - 1, N_DEV)
    right = lax.rem(my + 1, N_DEV)

    barrier = pltpu.get_barrier_semaphore()
    for nbr in (left, right):
        pl.semaphore_signal(
            barrier, inc=1, device_id=(nbr,), device_id_type=pl.DeviceIdType.MESH
        )
    pl.semaphore_wait(barrier, 2)

    comm_ref[0, 0] = k_ref[...]
    comm_ref[0, 1] = v_ref[...]

    for h in range(N_DEV - 1):
        rdma = pltpu.make_async_remote_copy(
            src_ref=comm_ref.at[h],
            dst_ref=comm_ref.at[h + 1],
            send_sem=send_sems.at[h],
            recv_sem=recv_sems.at[h],
            device_id=(right,),
            device_id_type=pl.DeviceIdType.MESH,
        )
        rdma.start()
        rdma.wait()

    n_heads, seq_q, d = q_ref.shape
    scale = d**-0.5
    for head in range(n_heads):
        q_h = q_ref[head]
        s_parts = []
        for slot in range(N_DEV):
            k_c = comm_ref[slot, 0, head]
            s_parts.append(
                lax.dot_general(
                    q_h,
                    k_c,
                    (((1,), (1,)), ((), ())),
                    preferred_element_type=jnp.float32,
                )
            )
        s = jnp.concatenate(s_parts, axis=1) * scale
        m = jnp.max(s, axis=1, keepdims=True)
        p = jnp.exp(s - m)
        denom = jnp.sum(p, axis=1, keepdims=True)
        p = (p / denom).astype(jnp.bfloat16)
        v_cat = jnp.concatenate(
            [comm_ref[slot, 1, head] for slot in range(N_DEV)], axis=0
        )
        out_ref[head] = lax.dot_general(
            p, v_cat, (((1,), (0,)), ((), ())), preferred_element_type=jnp.float32
        )


def kernel(Q, K, V):
    b, s, h, d = Q.shape
    q = jnp.transpose(Q[0].astype(jnp.bfloat16), (1, 0, 2))
    k = jnp.transpose(K[0].astype(jnp.bfloat16), (1, 0, 2))
    v = jnp.transpose(V[0].astype(jnp.bfloat16), (1, 0, 2))

    out = pl.pallas_call(
        _body,
        out_shape=jax.ShapeDtypeStruct((h, s, d), jnp.float32),
        in_specs=[pl.BlockSpec(memory_space=pltpu.VMEM)] * 3,
        out_specs=pl.BlockSpec(memory_space=pltpu.VMEM),
        scratch_shapes=[
            pltpu.VMEM((N_DEV, 2, h, s, d), jnp.bfloat16),
            pltpu.SemaphoreType.DMA((N_DEV - 1,)),
            pltpu.SemaphoreType.DMA((N_DEV - 1,)),
        ],
        compiler_params=pltpu.CompilerParams(collective_id=0),
    )(q, k, v)
    return jnp.transpose(out, (1, 0, 2))[None]


# baseline (device time: 418538 ns/iter reference)
import jax
import jax.numpy as jnp
from jax import lax
from jax.experimental import pallas as pl
from jax.experimental.pallas import tpu as pltpu

N_DEV = 4
N_HOPS = N_DEV - 1


def _body(q_ref, k_ref, v_ref, out_ref, comm_ref, send_sems, recv_sems):
    my = lax.axis_index("i")
    left = lax.rem(my + N_DEV - 1, N_DEV)
    right = lax.rem(my + 1, N_DEV)

    barrier = pltpu.get_barrier_semaphore()
    for nbr in (left, right):
        pl.semaphore_signal(
            barrier, inc=1, device_id=(nbr,), device_id_type=pl.DeviceIdType.MESH
        )
    pl.semaphore_wait(barrier, 2)

    def hop_rdma(h):
        if h == 0:
            return [
                pltpu.make_async_remote_copy(
                    src_ref=src,
                    dst_ref=comm_ref.at[0, i],
                    send_sem=send_sems.at[i],
                    recv_sem=recv_sems.at[i],
                    device_id=(right,),
                    device_id_type=pl.DeviceIdType.MESH,
                )
                for i, src in enumerate((k_ref, v_ref))
            ]
        return [
            pltpu.make_async_remote_copy(
                src_ref=comm_ref.at[h - 1],
                dst_ref=comm_ref.at[h],
                send_sem=send_sems.at[h + 1],
                recv_sem=recv_sems.at[h + 1],
                device_id=(right,),
                device_id_type=pl.DeviceIdType.MESH,
            )
        ]

    for h in range(N_HOPS):
        rdmas = hop_rdma(h)
        for r in rdmas:
            r.start()
        for r in rdmas:
            r.wait()

    n_heads, seq_q, d = q_ref.shape
    scale = d**-0.5

    def head_body(head, carry):
        q_h = q_ref[head]
        m = jnp.full((seq_q, 1), -1e30, jnp.float32)
        l = jnp.zeros((seq_q, 1), jnp.float32)
        acc = jnp.zeros((seq_q, d), jnp.float32)
        for c in range(N_DEV):
            if c == 0:
                k_c, v_c = k_ref[head], v_ref[head]
            else:
                k_c, v_c = comm_ref[c - 1, 0, head], comm_ref[c - 1, 1, head]
            s = (
                lax.dot_general(
                    q_h,
                    k_c,
                    (((1,), (1,)), ((), ())),
                    preferred_element_type=jnp.float32,
                )
                * scale
            )
            m_new = jnp.maximum(m, jnp.max(s, axis=1, keepdims=True))
            alpha = jnp.exp(m - m_new)
            p = jnp.exp(s - m_new)
            l = l * alpha + jnp.sum(p, axis=1, keepdims=True)
            acc = acc * alpha + lax.dot_general(
                p.astype(jnp.bfloat16),
                v_c,
                (((1,), (0,)), ((), ())),
                preferred_element_type=jnp.float32,
            )
            m = m_new
        out_ref[head] = acc / l
        return carry

    lax.fori_loop(0, n_heads, head_body, 0)


def kernel(Q, K, V):
    b, s, h, d = Q.shape
    q = jnp.transpose(Q[0].astype(jnp.bfloat16), (1, 0, 2))
    k = jnp.transpose(K[0].astype(jnp.bfloat16), (1, 0, 2))
    v = jnp.transpose(V[0].astype(jnp.bfloat16), (1, 0, 2))

    out = pl.pallas_call(
        _body,
        out_shape=jax.ShapeDtypeStruct((h, s, d), jnp.float32),
        in_specs=[pl.BlockSpec(memory_space=pltpu.VMEM)] * 3,
        out_specs=pl.BlockSpec(memory_space=pltpu.VMEM),
        scratch_shapes=[
            pltpu.VMEM((N_HOPS, 2, h, s, d), jnp.bfloat16),
            pltpu.SemaphoreType.DMA((N_HOPS + 1,)),
            pltpu.SemaphoreType.DMA((N_HOPS + 1,)),
        ],
        compiler_params=pltpu.CompilerParams(
            collective_id=0, vmem_limit_bytes=100 * 1024 * 1024
        ),
    )(q, k, v)
    return jnp.transpose(out, (1, 0, 2))[None]


# device time: 210422 ns/iter; 1.9890x vs baseline; 1.9890x over previous
import jax
import jax.numpy as jnp
from jax import lax
from jax.experimental import pallas as pl
from jax.experimental.pallas import tpu as pltpu

N_DEV = 4
N_HOPS = N_DEV - 1
N_HEADS = 16
H_HALF = N_HEADS // 2


def _body(
    q_ref, k_ref, v_ref, out_ref, commk_ref, commv_ref, m_ref, l_ref,
    send_sems, recv_sems,
):
    my = lax.axis_index("i")
    left = lax.rem(my + N_DEV - 1, N_DEV)
    right = lax.rem(my + 1, N_DEV)

    barrier = pltpu.get_barrier_semaphore()
    for nbr in (left, right):
        pl.semaphore_signal(
            barrier, inc=1, device_id=(nbr,), device_id_type=pl.DeviceIdType.MESH
        )
    pl.semaphore_wait(barrier, 2)

    def hop_rdmas(c):
        rs = []
        for d, tgt in ((0, right), (1, left)):
            hs = slice(0, H_HALF) if d == 0 else slice(H_HALF, N_HEADS)
            for t, (in_ref, comm) in enumerate(
                ((k_ref, commk_ref), (v_ref, commv_ref))
            ):
                src = in_ref.at[hs] if c == 0 else comm.at[c - 1, hs]
                idx = c * 4 + d * 2 + t
                rs.append(
                    pltpu.make_async_remote_copy(
                        src_ref=src,
                        dst_ref=comm.at[c, hs],
                        send_sem=send_sems.at[idx],
                        recv_sem=recv_sems.at[idx],
                        device_id=(tgt,),
                        device_id_type=pl.DeviceIdType.MESH,
                    )
                )
        return rs

    def compute_phase(c):

        def head_body(head, carry):
            q_t = q_ref[head]
            if c == 0:
                k_c, v_t = k_ref[head], v_ref[head]
            else:
                k_c, v_t = commk_ref[c - 1, head], commv_ref[c - 1, head]
            s_t = lax.dot_general(
                k_c, q_t, (((1,), (0,)), ((), ())),
                preferred_element_type=jnp.float32,
            )
            m_c = jnp.max(s_t, axis=0, keepdims=True)
            if c == 0:
                m_new = m_c
                p_t = jnp.exp(s_t - m_new)
                l_new = jnp.sum(p_t, axis=0, keepdims=True)
                acc = lax.dot_general(
                    v_t, p_t.astype(jnp.bfloat16),
                    (((1,), (0,)), ((), ())),
                    preferred_element_type=jnp.float32,
                )
            else:
                m_old = m_ref[head]
                l_old = l_ref[head]
                acc_old = out_ref[head]
                m_new = jnp.maximum(m_old, m_c)
                alpha = jnp.exp(m_old - m_new)
                p_t = jnp.exp(s_t - m_new)
                l_new = l_old * alpha + jnp.sum(p_t, axis=0, keepdims=True)
                acc = acc_old * alpha + lax.dot_general(
                    v_t, p_t.astype(jnp.bfloat16),
                    (((1,), (0,)), ((), ())),
                    preferred_element_type=jnp.float32,
                )
            if c == N_DEV - 1:
                out_ref[head] = acc / l_new
            else:
                out_ref[head] = acc
                m_ref[head] = m_new
                l_ref[head] = l_new
            return carry

        lax.fori_loop(0, N_HEADS, head_body, 0)

    all_rdmas = []
    prev = hop_rdmas(0)
    all_rdmas += prev
    for r in prev:
        r.start()
    compute_phase(0)
    for c in range(1, N_HOPS):
        for r in prev:
            r.wait_recv()
        prev = hop_rdmas(c)
        all_rdmas += prev
        for r in prev:
            r.start()
        compute_phase(c)
    for r in prev:
        r.wait_recv()
    compute_phase(N_DEV - 1)
    for r in all_rdmas:
        r.wait_send()


def kernel(Q, K, V):
    b, s, h, d = Q.shape
    scale = d**-0.5
    q = jnp.transpose((Q[0] * scale).astype(jnp.bfloat16), (1, 2, 0))
    k = jnp.transpose(K[0].astype(jnp.bfloat16), (1, 0, 2))
    v = jnp.transpose(V[0].astype(jnp.bfloat16), (1, 2, 0))

    out = pl.pallas_call(
        _body,
        out_shape=jax.ShapeDtypeStruct((h, d, s), jnp.float32),
        in_specs=[pl.BlockSpec(memory_space=pltpu.VMEM)] * 3,
        out_specs=pl.BlockSpec(memory_space=pltpu.VMEM),
        scratch_shapes=[
            pltpu.VMEM((N_HOPS, h, s, d), jnp.bfloat16),
            pltpu.VMEM((N_HOPS, h, d, s), jnp.bfloat16),
            pltpu.VMEM((h, 1, s), jnp.float32),
            pltpu.VMEM((h, 1, s), jnp.float32),
            pltpu.SemaphoreType.DMA((4 * N_HOPS,)),
            pltpu.SemaphoreType.DMA((4 * N_HOPS,)),
        ],
        compiler_params=pltpu.CompilerParams(
            collective_id=0, vmem_limit_bytes=100 * 1024 * 1024
        ),
    )(q, k, v)
    return jnp.transpose(out, (2, 0, 1))[None]


# device time: 197589 ns/iter; 2.1182x vs baseline; 1.0649x over previous
import jax
import jax.numpy as jnp
from jax import lax
from jax.experimental import pallas as pl
from jax.experimental.pallas import tpu as pltpu

N_DEV = 4
N_HOPS = N_DEV - 1
N_HEADS = 16
H_HALF = N_HEADS // 2


def _body(
    q_ref, k_ref, v_ref, out_ref, commk_ref, commv_ref, m_ref, l_ref,
    send_sems, recv_sems,
):
    my = lax.axis_index("i")
    left = lax.rem(my + N_DEV - 1, N_DEV)
    right = lax.rem(my + 1, N_DEV)

    barrier = pltpu.get_barrier_semaphore()
    for nbr in (left, right):
        pl.semaphore_signal(
            barrier, inc=1, device_id=(nbr,), device_id_type=pl.DeviceIdType.MESH
        )
    pl.semaphore_wait(barrier, 2)

    def hop_rdmas(c):
        rs = []
        for d, tgt in ((0, right), (1, left)):
            hs = slice(0, H_HALF) if d == 0 else slice(H_HALF, N_HEADS)
            for t, (in_ref, comm) in enumerate(
                ((k_ref, commk_ref), (v_ref, commv_ref))
            ):
                src = in_ref.at[hs] if c == 0 else comm.at[c - 1, hs]
                idx = c * 4 + d * 2 + t
                rs.append(
                    pltpu.make_async_remote_copy(
                        src_ref=src,
                        dst_ref=comm.at[c, hs],
                        send_sem=send_sems.at[idx],
                        recv_sem=recv_sems.at[idx],
                        device_id=(tgt,),
                        device_id_type=pl.DeviceIdType.MESH,
                    )
                )
        return rs

    def compute_phase(c):

        def head_body(head, carry):
            q_t = q_ref[head]
            if c == 0:
                k_c, v_t = k_ref[head], v_ref[head]
            else:
                k_c, v_t = commk_ref[c - 1, head], commv_ref[c - 1, head]
            s_t = lax.dot_general(
                k_c, q_t, (((1,), (0,)), ((), ())),
                preferred_element_type=jnp.float32,
            )
            if c == 0:
                m = jnp.max(s_t, axis=0, keepdims=True)
                m_ref[head] = m
            else:
                m = m_ref[head]
            p_t = jnp.exp2(s_t - m)
            l_c = jnp.sum(p_t, axis=0, keepdims=True)
            pv = lax.dot_general(
                v_t, p_t.astype(jnp.bfloat16),
                (((1,), (0,)), ((), ())),
                preferred_element_type=jnp.float32,
            )
            if c == 0:
                l_new = l_c
                acc = pv
            else:
                l_new = l_ref[head] + l_c
                acc = out_ref[head] + pv
            if c == N_DEV - 1:
                out_ref[head] = acc / l_new
            else:
                out_ref[head] = acc
                l_ref[head] = l_new
            return carry

        lax.fori_loop(0, N_HEADS, head_body, 0)

    all_rdmas = []
    prev = hop_rdmas(0)
    all_rdmas += prev
    for r in prev:
        r.start()
    compute_phase(0)
    for c in range(1, N_HOPS):
        for r in prev:
            r.wait_recv()
        prev = hop_rdmas(c)
        all_rdmas += prev
        for r in prev:
            r.start()
        compute_phase(c)
    for r in prev:
        r.wait_recv()
    compute_phase(N_DEV - 1)
    for r in all_rdmas:
        r.wait_send()


def kernel(Q, K, V):
    b, s, h, d = Q.shape
    scale = d**-0.5 * 1.4426950408889634
    q = jnp.transpose((Q[0] * scale).astype(jnp.bfloat16), (1, 2, 0))
    k = jnp.transpose(K[0].astype(jnp.bfloat16), (1, 0, 2))
    v = jnp.transpose(V[0].astype(jnp.bfloat16), (1, 2, 0))

    out = pl.pallas_call(
        _body,
        out_shape=jax.ShapeDtypeStruct((h, d, s), jnp.float32),
        in_specs=[pl.BlockSpec(memory_space=pltpu.VMEM)] * 3,
        out_specs=pl.BlockSpec(memory_space=pltpu.VMEM),
        scratch_shapes=[
            pltpu.VMEM((N_HOPS, h, s, d), jnp.bfloat16),
            pltpu.VMEM((N_HOPS, h, d, s), jnp.bfloat16),
            pltpu.VMEM((h, 1, s), jnp.float32),
            pltpu.VMEM((h, 1, s), jnp.float32),
            pltpu.SemaphoreType.DMA((4 * N_HOPS,)),
            pltpu.SemaphoreType.DMA((4 * N_HOPS,)),
        ],
        compiler_params=pltpu.CompilerParams(
            collective_id=0, vmem_limit_bytes=100 * 1024 * 1024
        ),
    )(q, k, v)
    return jnp.transpose(out, (2, 0, 1))[None]
